# baseline (speedup 1.0000x reference)
import functools

import jax
import jax.numpy as jnp
from jax import lax
from jax.experimental import pallas as pl
from jax.experimental.pallas import tpu as pltpu
from jax.experimental.pallas import tpu_sc as plsc

_N = 4096
_T = 20
_E = 128
_H = 128
_O = 32


def _gather_rows_sc(embed, idx):
    V, D = embed.shape
    (B,) = idx.shape
    info = plsc.get_sparse_core_info()
    nw = info.num_cores * info.num_subcores
    b_per_w = B // nw
    ch = 256
    n_ch = b_per_w // ch
    mesh = plsc.VectorSubcoreMesh(core_axis_name="c", subcore_axis_name="s")

    @functools.partial(
        pl.kernel,
        out_type=jax.ShapeDtypeStruct((B, D), jnp.float32),
        mesh=mesh,
        scratch_types=[
            pltpu.VMEM((b_per_w,), jnp.int32),
            pltpu.VMEM((2, ch, D), jnp.float32),
            pltpu.SemaphoreType.DMA,
            pltpu.SemaphoreType.DMA,
            pltpu.SemaphoreType.DMA,
        ],
    )
    def k(table_hbm, idx_hbm, out_hbm, idx_v, rows_v, gsem, ssem0, ssem1):
        wid = lax.axis_index("s") * info.num_cores + lax.axis_index("c")
        base = wid * b_per_w
        pltpu.sync_copy(idx_hbm.at[pl.ds(base, b_per_w)], idx_v)
        ssems = (ssem0, ssem1)
        scats = [None, None]
        for i in range(n_ch):
            bf = i % 2
            if scats[bf] is not None:
                scats[bf].wait()
            pltpu.async_copy(
                table_hbm.at[idx_v.at[pl.ds(i * ch, ch)]], rows_v.at[bf], gsem
            ).wait()
            scats[bf] = pltpu.async_copy(
                rows_v.at[bf], out_hbm.at[pl.ds(base + i * ch, ch)], ssems[bf]
            )
        scats[(n_ch - 1) % 2].wait()
        scats[n_ch % 2].wait()

    return k(embed, idx)


def _sig(v):
    return 0.5 * jnp.tanh(0.5 * v) + 0.5


def _lstm_step(x_ref, wc_ref, b_ref, z_ref, c_ref):
    z_ref[:, :_E] = x_ref[0].astype(jnp.bfloat16)
    gates = jnp.dot(z_ref[...], wc_ref[...], preferred_element_type=jnp.float32)
    gates = gates + b_ref[...]
    i = _sig(gates[:, 0 * _H:1 * _H])
    f = _sig(gates[:, 1 * _H:2 * _H])
    g = jnp.tanh(gates[:, 2 * _H:3 * _H])
    o = _sig(gates[:, 3 * _H:4 * _H])
    c = f * c_ref[...] + i * g
    h = o * jnp.tanh(c)
    c_ref[...] = c
    hb = h.astype(jnp.bfloat16)
    z_ref[:, _E:] = hb
    return hb


def _lstm_body_a(x_ref, wc_ref, b_ref, hb_out, c_out, z_ref, c_ref):
    t = pl.program_id(0)

    @pl.when(t == 0)
    def _():
        z_ref[:, _E:] = jnp.zeros_like(z_ref[:, _E:])
        c_ref[...] = jnp.zeros_like(c_ref)

    hb = _lstm_step(x_ref, wc_ref, b_ref, z_ref, c_ref)

    @pl.when(t == pl.num_programs(0) - 1)
    def _():
        hb_out[...] = hb
        c_out[...] = c_ref[...]


def _lstm_body_b(x_ref, wc_ref, b_ref, hb0_ref, c0_ref, w1_ref, out_ref,
                 z_ref, c_ref):
    t = pl.program_id(0)

    @pl.when(t == 0)
    def _():
        z_ref[:, _E:] = hb0_ref[...]
        c_ref[...] = c0_ref[...]

    hb = _lstm_step(x_ref, wc_ref, b_ref, z_ref, c_ref)

    @pl.when(t == pl.num_programs(0) - 1)
    def _():
        out_ref[...] = jnp.dot(hb, w1_ref[...], preferred_element_type=jnp.float32)


def _lstm_phase_a(x, wc, b):
    t0 = x.shape[0]
    return pl.pallas_call(
        _lstm_body_a,
        grid=(t0,),
        in_specs=[
            pl.BlockSpec((1, _N, _E), lambda t: (t, 0, 0)),
            pl.BlockSpec((_E + _H, 4 * _H), lambda t: (0, 0)),
            pl.BlockSpec((1, 4 * _H), lambda t: (0, 0)),
        ],
        out_specs=[
            pl.BlockSpec((_N, _H), lambda t: (0, 0)),
            pl.BlockSpec((_N, _H), lambda t: (0, 0)),
        ],
        out_shape=[
            jax.ShapeDtypeStruct((_N, _H), jnp.bfloat16),
            jax.ShapeDtypeStruct((_N, _H), jnp.float32),
        ],
        scratch_shapes=[
            pltpu.VMEM((_N, _E + _H), jnp.bfloat16),
            pltpu.VMEM((_N, _H), jnp.float32),
        ],
    )(x, wc, b)


def _lstm_phase_b(x, wc, b, hb0, c0, W1bf):
    t0 = x.shape[0]
    return pl.pallas_call(
        _lstm_body_b,
        grid=(t0,),
        in_specs=[
            pl.BlockSpec((1, _N, _E), lambda t: (t, 0, 0)),
            pl.BlockSpec((_E + _H, 4 * _H), lambda t: (0, 0)),
            pl.BlockSpec((1, 4 * _H), lambda t: (0, 0)),
            pl.BlockSpec((_N, _H), lambda t: (0, 0)),
            pl.BlockSpec((_N, _H), lambda t: (0, 0)),
            pl.BlockSpec((_H, 2 * _H), lambda t: (0, 0)),
        ],
        out_specs=pl.BlockSpec((_N, 2 * _H), lambda t: (0, 0)),
        out_shape=jax.ShapeDtypeStruct((_N, 2 * _H), jnp.float32),
        scratch_shapes=[
            pltpu.VMEM((_N, _E + _H), jnp.bfloat16),
            pltpu.VMEM((_N, _H), jnp.float32),
        ],
    )(x, wc, b, hb0, c0, W1bf)


def _gcn1_body(adj_ref, s1_ref, w2_ref, b1_ref, out_ref):
    s = s1_ref[...].astype(jnp.bfloat16)
    t = jnp.dot(adj_ref[...], s, preferred_element_type=jnp.float32)
    t = jnp.maximum(t + b1_ref[...], 0.0)
    out_ref[...] = jnp.dot(t.astype(jnp.bfloat16), w2_ref[...],
                           preferred_element_type=jnp.float32)


def _gcn1(adj_bf, s1, W2bf, b1, bm):
    return pl.pallas_call(
        _gcn1_body,
        grid=(_N // bm,),
        in_specs=[
            pl.BlockSpec((bm, _N), lambda i: (i, 0)),
            pl.BlockSpec((_N, 2 * _H), lambda i: (0, 0)),
            pl.BlockSpec((2 * _H, _O), lambda i: (0, 0)),
            pl.BlockSpec((1, 2 * _H), lambda i: (0, 0)),
        ],
        out_specs=pl.BlockSpec((bm, _O), lambda i: (i, 0)),
        out_shape=jax.ShapeDtypeStruct((_N, _O), jnp.float32),
    )(adj_bf, s1, W2bf, b1)


def _gcn2_body(adj_ref, s2_ref, b2_ref, out_ref):
    s = s2_ref[...].astype(jnp.bfloat16)
    y = jnp.dot(adj_ref[...], s, preferred_element_type=jnp.float32)
    y = y + b2_ref[...]
    m = jnp.max(y, axis=1, keepdims=True)
    y = y - m
    lse = jnp.log(jnp.sum(jnp.exp(y), axis=1, keepdims=True))
    out_ref[...] = y - lse


def _gcn2(adj_bf, s2, b2, bm):
    return pl.pallas_call(
        _gcn2_body,
        grid=(_N // bm,),
        in_specs=[
            pl.BlockSpec((bm, _N), lambda i: (i, 0)),
            pl.BlockSpec((_N, _O), lambda i: (0, 0)),
            pl.BlockSpec((1, _O), lambda i: (0, 0)),
        ],
        out_specs=pl.BlockSpec((bm, _O), lambda i: (i, 0)),
        out_shape=jax.ShapeDtypeStruct((_N, _O), jnp.float32),
    )(adj_bf, s2, b2)


def kernel(inputs, adj, embed, W_ih, W_hh, b_ih, b_hh, W1, b1, W2, b2):
    adj_bf = adj.astype(jnp.bfloat16)
    idx = jnp.transpose(inputs).reshape(-1).astype(jnp.int32)
    half = (_T // 2) * _N
    x1 = _gather_rows_sc(embed, idx[:half]).reshape(_T // 2, _N, _E)
    x2 = _gather_rows_sc(embed, idx[half:]).reshape(_T - _T // 2, _N, _E)

    b = (b_ih + b_hh).reshape(1, 4 * _H)
    wc = jnp.concatenate([W_ih.T, W_hh.T], axis=0).astype(jnp.bfloat16)
    hb0, c0 = _lstm_phase_a(x1, wc, b)
    support1 = _lstm_phase_b(x2, wc, b, hb0, c0, W1.astype(jnp.bfloat16))
    support2 = _gcn1(adj_bf, support1, W2.astype(jnp.bfloat16),
                     b1.reshape(1, 2 * _H), 512)
    return _gcn2(adj_bf, support2, b2.reshape(1, _O), 512)


# speedup vs baseline: 1.1385x; 1.1385x over previous
import functools

import jax
import jax.numpy as jnp
from jax import lax
from jax.experimental import pallas as pl
from jax.experimental.pallas import tpu as pltpu
from jax.experimental.pallas import tpu_sc as plsc

_N = 4096
_T = 20
_E = 128
_H = 128
_O = 32


def _gather_rows_sc(embed, idx):
    V, D = embed.shape
    (B,) = idx.shape
    info = plsc.get_sparse_core_info()
    nw = info.num_cores * info.num_subcores
    b_per_w = B // nw
    ch = 320
    nb = 3
    n_ch = b_per_w // ch
    mesh = plsc.VectorSubcoreMesh(core_axis_name="c", subcore_axis_name="s")

    @functools.partial(
        pl.kernel,
        out_type=jax.ShapeDtypeStruct((B, D), jnp.float32),
        mesh=mesh,
        scratch_types=[
            pltpu.VMEM((b_per_w,), jnp.int32),
            pltpu.VMEM((nb, ch, D), jnp.float32),
            [pltpu.SemaphoreType.DMA] * nb,
            [pltpu.SemaphoreType.DMA] * nb,
        ],
    )
    def k(table_hbm, idx_hbm, out_hbm, idx_v, rows_v, gsems, ssems):
        wid = lax.axis_index("s") * info.num_cores + lax.axis_index("c")
        base = wid * b_per_w
        pltpu.sync_copy(idx_hbm.at[pl.ds(base, b_per_w)], idx_v)

        def gather(i):
            return pltpu.async_copy(
                table_hbm.at[idx_v.at[pl.ds(i * ch, ch)]],
                rows_v.at[i % nb], gsems[i % nb],
            )

        gat = [None] * n_ch
        sca = [None] * n_ch
        for i in range(min(2, n_ch)):
            gat[i] = gather(i)
        for i in range(n_ch):
            b = i % nb
            gat[i].wait()
            sca[i] = pltpu.async_copy(
                rows_v.at[b], out_hbm.at[pl.ds(base + i * ch, ch)], ssems[b]
            )
            j = i + 2
            if j < n_ch:
                if sca[j - nb] is not None:
                    sca[j - nb].wait()
                gat[j] = gather(j)
        for i in range(max(0, n_ch - nb), n_ch):
            sca[i].wait()

    return k(embed, idx)


def _sig(v):
    return 0.5 * jnp.tanh(0.5 * v) + 0.5


def _lstm_step(x_ref, wc_ref, b_ref, z_ref, c_ref):
    z_ref[:, :_E] = x_ref[0].astype(jnp.bfloat16)
    gates = jnp.dot(z_ref[...], wc_ref[...], preferred_element_type=jnp.float32)
    gates = gates + b_ref[...]
    i = _sig(gates[:, 0 * _H:1 * _H])
    f = _sig(gates[:, 1 * _H:2 * _H])
    g = jnp.tanh(gates[:, 2 * _H:3 * _H])
    o = _sig(gates[:, 3 * _H:4 * _H])
    c = f * c_ref[...] + i * g
    h = o * jnp.tanh(c)
    c_ref[...] = c
    hb = h.astype(jnp.bfloat16)
    z_ref[:, _E:] = hb
    return hb


def _lstm_body(x_ref, wc_ref, b_ref, w1_ref, out_ref, z_ref, c_ref):
    t = pl.program_id(0)

    @pl.when(t == 0)
    def _():
        z_ref[:, _E:] = jnp.zeros_like(z_ref[:, _E:])
        c_ref[...] = jnp.zeros_like(c_ref)

    hb = _lstm_step(x_ref, wc_ref, b_ref, z_ref, c_ref)

    @pl.when(t == pl.num_programs(0) - 1)
    def _():
        out_ref[...] = jnp.dot(hb, w1_ref[...], preferred_element_type=jnp.float32)


def _lstm(x, wc, b, W1bf):
    return pl.pallas_call(
        _lstm_body,
        grid=(_T,),
        in_specs=[
            pl.BlockSpec((1, _N, _E), lambda t: (t, 0, 0)),
            pl.BlockSpec((_E + _H, 4 * _H), lambda t: (0, 0)),
            pl.BlockSpec((1, 4 * _H), lambda t: (0, 0)),
            pl.BlockSpec((_H, 2 * _H), lambda t: (0, 0)),
        ],
        out_specs=pl.BlockSpec((_N, 2 * _H), lambda t: (0, 0)),
        out_shape=jax.ShapeDtypeStruct((_N, 2 * _H), jnp.float32),
        scratch_shapes=[
            pltpu.VMEM((_N, _E + _H), jnp.bfloat16),
            pltpu.VMEM((_N, _H), jnp.float32),
        ],
    )(x, wc, b, W1bf)


def _gcn1_body(adj_ref, s1_ref, w2_ref, b1_ref, out_ref):
    s = s1_ref[...].astype(jnp.bfloat16)
    t = jnp.dot(adj_ref[...].astype(jnp.bfloat16), s,
                preferred_element_type=jnp.float32)
    t = jnp.maximum(t + b1_ref[...], 0.0)
    out_ref[...] = jnp.dot(t.astype(jnp.bfloat16), w2_ref[...],
                           preferred_element_type=jnp.float32)


def _gcn1(adj_bf, s1, W2bf, b1, bm):
    return pl.pallas_call(
        _gcn1_body,
        grid=(_N // bm,),
        in_specs=[
            pl.BlockSpec((bm, _N), lambda i: (i, 0)),
            pl.BlockSpec((_N, 2 * _H), lambda i: (0, 0)),
            pl.BlockSpec((2 * _H, _O), lambda i: (0, 0)),
            pl.BlockSpec((1, 2 * _H), lambda i: (0, 0)),
        ],
        out_specs=pl.BlockSpec((bm, _O), lambda i: (i, 0)),
        out_shape=jax.ShapeDtypeStruct((_N, _O), jnp.float32),
    )(adj_bf, s1, W2bf, b1)


def _gcn2_body(adj_ref, s2_ref, b2_ref, out_ref):
    s = s2_ref[...].astype(jnp.bfloat16)
    y = jnp.dot(adj_ref[...].astype(jnp.bfloat16), s,
                preferred_element_type=jnp.float32)
    y = y + b2_ref[...]
    m = jnp.max(y, axis=1, keepdims=True)
    y = y - m
    lse = jnp.log(jnp.sum(jnp.exp(y), axis=1, keepdims=True))
    out_ref[...] = y - lse


def _gcn2(adj_bf, s2, b2, bm):
    return pl.pallas_call(
        _gcn2_body,
        grid=(_N // bm,),
        in_specs=[
            pl.BlockSpec((bm, _N), lambda i: (i, 0)),
            pl.BlockSpec((_N, _O), lambda i: (0, 0)),
            pl.BlockSpec((1, _O), lambda i: (0, 0)),
        ],
        out_specs=pl.BlockSpec((bm, _O), lambda i: (i, 0)),
        out_shape=jax.ShapeDtypeStruct((_N, _O), jnp.float32),
    )(adj_bf, s2, b2)


def kernel(inputs, adj, embed, W_ih, W_hh, b_ih, b_hh, W1, b1, W2, b2):
    idx = jnp.transpose(inputs).reshape(-1).astype(jnp.int32)
    x = _gather_rows_sc(embed, idx).reshape(_T, _N, _E)

    b = (b_ih + b_hh).reshape(1, 4 * _H)
    wc = jnp.concatenate([W_ih.T, W_hh.T], axis=0).astype(jnp.bfloat16)
    support1 = _lstm(x, wc, b, W1.astype(jnp.bfloat16))
    support2 = _gcn1(adj, support1, W2.astype(jnp.bfloat16),
                     b1.reshape(1, 2 * _H), 512)
    return _gcn2(adj, support2, b2.reshape(1, _O), 512)
